# Initial kernel scaffold; baseline (speedup 1.0000x reference)
import jax
import jax.numpy as jnp
from jax.experimental import pallas as pl


def kernel(vertices, feature_map, neighbor_index_density):
    raise NotImplementedError("write your pallas kernel here")


# speedup vs baseline: 2.5936x; 2.5936x over previous
import functools

import numpy as np
import jax
import jax.numpy as jnp
from jax import lax
from jax.experimental import pallas as pl
from jax.experimental.pallas import tpu as pltpu
from jax.experimental.pallas import tpu_sc as plsc

_BS, _V, _C, _NB, _POOL_RATE, _NUM_BINS, _DENS_NB = 8, 2048, 256, 16, 4, 3, 20
_TS = _V // _POOL_RATE


def _gumbel_table():
    skey = jax.random.key(42)
    gs = [jax.random.gumbel(jax.random.fold_in(skey, t), (_V,))
          for t in range(_BS * _NUM_BINS)]
    return jnp.stack(gs).reshape(_BS, _NUM_BINS, _V)


def _sample_indices(vertices, neighbor_index_density):
    id0 = jnp.arange(_BS)[:, None, None]
    neighbor_to_vertices = vertices[id0, neighbor_index_density]
    expanded_vertices = jnp.broadcast_to(
        vertices[:, :, None, :], (_BS, _V, _DENS_NB, 3))
    squared_diff = (expanded_vertices - neighbor_to_vertices) ** 2
    distances = jnp.sqrt(jnp.sum(squared_diff, axis=3))
    summed = jnp.sum(distances, axis=2)
    mn = jnp.min(summed, axis=1, keepdims=True)
    mx = jnp.max(summed, axis=1, keepdims=True)
    nd = (summed - mn) / (mx - mn)

    edges = jnp.linspace(0.0, 1.0, _NUM_BINS + 1)
    bidx = jnp.searchsorted(edges, nd, side='left')
    counts = jnp.stack(
        [jnp.sum(bidx == b, axis=1) for b in range(1, _NUM_BINS + 1)],
        axis=1).astype(jnp.float32)
    bin_samples = counts / jnp.sum(counts, axis=1, keepdims=True) * _TS
    s = jnp.floor(bin_samples).astype(jnp.int32)

    g = _gumbel_table()
    mask = bidx[:, None, :] == jnp.arange(1, _NUM_BINS + 1)[None, :, None]
    scores = jnp.where(mask, g, -jnp.inf)
    _, chosen = lax.top_k(scores, _TS)

    cnt = jnp.concatenate(
        [jnp.zeros((_BS, 1), jnp.int32), jnp.cumsum(s, axis=1)[:, :2]], axis=1)
    pos = jnp.arange(_TS)
    wpos = jnp.where(pos[None, None, :] < s[:, :, None],
                     cnt[:, :, None] + pos[None, None, :], _TS)
    bi = jnp.broadcast_to(jnp.arange(_BS)[:, None, None], wpos.shape)
    sel = (jnp.zeros((_BS, _TS), jnp.int32)
           .at[bi, wpos].set(chosen.astype(jnp.int32), mode='drop'))
    return sel


def _knn_body(dist_ref, out_ref):
    dist = dist_ref[0]
    col = lax.broadcasted_iota(jnp.int32, (_TS, _V), 1)
    inf = jnp.float32(jnp.inf)
    b = pl.program_id(0)
    for k in range(_NB + 1):
        m = jnp.min(dist, axis=1, keepdims=True)
        idx = jnp.min(jnp.where(dist == m, col, _V), axis=1, keepdims=True)
        if k > 0:
            out_ref[0, :, pl.ds(k - 1, 1)] = idx + b * _V
        dist = jnp.where(col == idx, inf, dist)


def _knn(dist):
    return pl.pallas_call(
        _knn_body,
        grid=(_BS,),
        in_specs=[
            pl.BlockSpec((1, _TS, _V), lambda b: (b, 0, 0)),
        ],
        out_specs=pl.BlockSpec((1, _TS, _NB), lambda b: (b, 0, 0)),
        out_shape=jax.ShapeDtypeStruct((_BS, _TS, _NB), jnp.int32),
    )(dist)


def _pool_gather(fm_flat, ids_flat):
    NC, NS = 2, 16
    NW = NC * NS
    ROWS = _BS * _TS
    RPW = ROWS // NW
    CH = 8
    NCH = RPW // CH
    G = CH * _NB

    mesh = plsc.VectorSubcoreMesh(core_axis_name="c", subcore_axis_name="s")

    @functools.partial(
        pl.kernel,
        mesh=mesh,
        out_type=jax.ShapeDtypeStruct((ROWS, _C), jnp.float32),
        scratch_types=[
            pltpu.VMEM((G,), jnp.int32),
            pltpu.VMEM((G, _C), jnp.float32),
            pltpu.VMEM((CH, _C), jnp.float32),
            pltpu.SemaphoreType.DMA,
        ],
    )
    def pool(fm_hbm, ids_hbm, out_hbm, idx_v, rows_v, out_v, sem):
        wid = lax.axis_index("s") * NC + lax.axis_index("c")

        def chunk_body(k, carry):
            base = (wid * RPW + k * CH) * _NB
            pltpu.sync_copy(ids_hbm.at[pl.ds(base, G)], idx_v)
            pltpu.async_copy(fm_hbm.at[idx_v], rows_v, sem).wait()

            def row_body(r, carry2):
                for cc in range(_C // 16):
                    sl = pl.ds(cc * 16, 16)
                    acc = rows_v[r * _NB, sl]
                    for j in range(1, _NB):
                        acc = jnp.maximum(acc, rows_v[r * _NB + j, sl])
                    out_v[r, sl] = acc
                return carry2

            lax.fori_loop(0, CH, row_body, 0)
            pltpu.sync_copy(out_v, out_hbm.at[pl.ds(wid * RPW + k * CH, CH)])
            return carry

        lax.fori_loop(0, NCH, chunk_body, 0)

    return pool(fm_flat, ids_flat)


def kernel(vertices, feature_map, neighbor_index_density):
    sel = _sample_indices(vertices, neighbor_index_density)
    bi = jnp.arange(_BS)[:, None]
    vertices_pool = vertices[bi, sel]
    inner = jnp.einsum('bvd,bwd->bvw', vertices, vertices)
    quadratic = jnp.sum(vertices ** 2, axis=2)
    inner_sel = inner[bi, sel]
    q_sel = quadratic[bi, sel]
    dist = inner_sel * -2 + quadratic[:, None, :] + q_sel[:, :, None]
    gids = _knn(dist)
    fm_flat = feature_map.reshape(_BS * _V, _C)
    pooled = _pool_gather(fm_flat, gids.reshape(-1))
    return vertices_pool, pooled.reshape(_BS, _TS, _C)


# speedup vs baseline: 9.8385x; 3.7934x over previous
import functools

import numpy as np
import jax
import jax.numpy as jnp
from jax import lax
from jax.experimental import pallas as pl
from jax.experimental.pallas import tpu as pltpu
from jax.experimental.pallas import tpu_sc as plsc

_BS, _V, _C, _NB, _POOL_RATE, _NUM_BINS, _DENS_NB = 8, 2048, 256, 16, 4, 3, 20
_TS = _V // _POOL_RATE


def _gumbel_table():
    skey = jax.random.key(42)
    gs = [jax.random.gumbel(jax.random.fold_in(skey, t), (_V,))
          for t in range(_BS * _NUM_BINS)]
    return jnp.stack(gs).reshape(_BS, _NUM_BINS, _V)


def _sample_indices(vertices, neighbor_index_density):
    vflat = vertices.reshape(_BS * _V, 3)
    gidx = (neighbor_index_density.astype(jnp.int32)
            + (jnp.arange(_BS, dtype=jnp.int32) * _V)[:, None, None])
    neighbor_to_vertices = vflat[gidx.reshape(-1)].reshape(_BS, _V, _DENS_NB, 3)
    expanded_vertices = jnp.broadcast_to(
        vertices[:, :, None, :], (_BS, _V, _DENS_NB, 3))
    squared_diff = (expanded_vertices - neighbor_to_vertices) ** 2
    distances = jnp.sqrt(jnp.sum(squared_diff, axis=3))
    summed = jnp.sum(distances, axis=2)
    mn = jnp.min(summed, axis=1, keepdims=True)
    mx = jnp.max(summed, axis=1, keepdims=True)
    nd = (summed - mn) / (mx - mn)

    edges = jnp.linspace(0.0, 1.0, _NUM_BINS + 1)
    bidx = ((nd > edges[0]).astype(jnp.int32)
            + (nd > edges[1]).astype(jnp.int32)
            + (nd > edges[2]).astype(jnp.int32))
    counts = jnp.stack(
        [jnp.sum(bidx == b, axis=1) for b in range(1, _NUM_BINS + 1)],
        axis=1).astype(jnp.float32)
    bin_samples = counts / jnp.sum(counts, axis=1, keepdims=True) * _TS
    s = jnp.floor(bin_samples).astype(jnp.int32)

    g = _gumbel_table()
    mask = bidx[:, None, :] == jnp.arange(1, _NUM_BINS + 1)[None, :, None]
    scores = jnp.where(mask, g, -jnp.inf)
    _, chosen = lax.top_k(scores, _TS)

    cnt = jnp.concatenate(
        [jnp.zeros((_BS, 1), jnp.int32), jnp.cumsum(s, axis=1)[:, :2]], axis=1)
    pos = jnp.arange(_TS)
    wpos = jnp.where(pos[None, None, :] < s[:, :, None],
                     cnt[:, :, None] + pos[None, None, :], _TS)
    bi = jnp.broadcast_to(jnp.arange(_BS)[:, None, None], wpos.shape)
    sel = (jnp.zeros((_BS, _TS), jnp.int32)
           .at[bi, wpos].set(chosen.astype(jnp.int32), mode='drop'))
    return sel


def _knn_body(dist_ref, out_ref):
    dist = dist_ref[0]
    col = lax.broadcasted_iota(jnp.int32, (_TS, _V), 1)
    inf = jnp.float32(jnp.inf)
    b = pl.program_id(0)
    for k in range(_NB + 1):
        m = jnp.min(dist, axis=1, keepdims=True)
        idx = jnp.min(jnp.where(dist == m, col, _V), axis=1, keepdims=True)
        if k > 0:
            out_ref[0, :, pl.ds(k - 1, 1)] = idx + b * _V
        dist = jnp.where(col == idx, inf, dist)


def _knn(dist):
    return pl.pallas_call(
        _knn_body,
        grid=(_BS,),
        in_specs=[
            pl.BlockSpec((1, _TS, _V), lambda b: (b, 0, 0)),
        ],
        out_specs=pl.BlockSpec((1, _TS, _NB), lambda b: (b, 0, 0)),
        out_shape=jax.ShapeDtypeStruct((_BS, _TS, _NB), jnp.int32),
    )(dist)


def _pool_gather(fm_flat, ids_flat):
    NC, NS = 2, 16
    NW = NC * NS
    ROWS = _BS * _TS
    RPW = ROWS // NW
    CH = 8
    NCH = RPW // CH
    G = CH * _NB

    mesh = plsc.VectorSubcoreMesh(core_axis_name="c", subcore_axis_name="s")

    @functools.partial(
        pl.kernel,
        mesh=mesh,
        out_type=jax.ShapeDtypeStruct((ROWS, _C), jnp.float32),
        scratch_types=[
            pltpu.VMEM((G,), jnp.int32),
            pltpu.VMEM((G, _C), jnp.float32),
            pltpu.VMEM((CH, _C), jnp.float32),
            pltpu.SemaphoreType.DMA,
        ],
    )
    def pool(fm_hbm, ids_hbm, out_hbm, idx_v, rows_v, out_v, sem):
        wid = lax.axis_index("s") * NC + lax.axis_index("c")

        def chunk_body(k, carry):
            base = (wid * RPW + k * CH) * _NB
            pltpu.sync_copy(ids_hbm.at[pl.ds(base, G)], idx_v)
            pltpu.async_copy(fm_hbm.at[idx_v], rows_v, sem).wait()

            def row_body(r, carry2):
                for cc in range(_C // 16):
                    sl = pl.ds(cc * 16, 16)
                    acc = rows_v[r * _NB, sl]
                    for j in range(1, _NB):
                        acc = jnp.maximum(acc, rows_v[r * _NB + j, sl])
                    out_v[r, sl] = acc
                return carry2

            lax.fori_loop(0, CH, row_body, 0)
            pltpu.sync_copy(out_v, out_hbm.at[pl.ds(wid * RPW + k * CH, CH)])
            return carry

        lax.fori_loop(0, NCH, chunk_body, 0)

    return pool(fm_flat, ids_flat)


def kernel(vertices, feature_map, neighbor_index_density):
    sel = _sample_indices(vertices, neighbor_index_density)
    bi = jnp.arange(_BS)[:, None]
    vertices_pool = vertices[bi, sel]
    inner = jnp.einsum('bvd,bwd->bvw', vertices, vertices)
    quadratic = jnp.sum(vertices ** 2, axis=2)
    inner_sel = inner[bi, sel]
    q_sel = quadratic[bi, sel]
    dist = inner_sel * -2 + quadratic[:, None, :] + q_sel[:, :, None]
    gids = _knn(dist)
    fm_flat = feature_map.reshape(_BS * _V, _C)
    pooled = _pool_gather(fm_flat, gids.reshape(-1))
    return vertices_pool, pooled.reshape(_BS, _TS, _C)


# speedup vs baseline: 19.7635x; 2.0088x over previous
import functools

import numpy as np
import jax
import jax.numpy as jnp
from jax import lax
from jax.experimental import pallas as pl
from jax.experimental.pallas import tpu as pltpu
from jax.experimental.pallas import tpu_sc as plsc

_BS, _V, _C, _NB, _POOL_RATE, _NUM_BINS, _DENS_NB = 8, 2048, 256, 16, 4, 3, 20
_TS = _V // _POOL_RATE


def _gumbel_table():
    skey = jax.random.key(42)
    gs = [jax.random.gumbel(jax.random.fold_in(skey, t), (_V,))
          for t in range(_BS * _NUM_BINS)]
    return jnp.stack(gs).reshape(_BS, _NUM_BINS, _V)


def _dens_gather(vflat, nid_g):
    WPB = 4
    VPW = _V // WPB
    IDS = VPW * _DENS_NB
    NCHUNK = IDS // 128
    FIRE = 4

    mesh = plsc.VectorSubcoreMesh(core_axis_name="c", subcore_axis_name="s")

    @functools.partial(
        pl.kernel, mesh=mesh,
        out_type=jax.ShapeDtypeStruct((_BS, _V * _DENS_NB, 128), jnp.float32),
        scratch_types=[
            pltpu.VMEM((NCHUNK, 128), jnp.int32),
            pltpu.VMEM((FIRE, 128, 128), jnp.float32),
            pltpu.SemaphoreType.DMA,
        ],
    )
    def dens(v_hbm, nid_hbm, out_hbm, idx_v, buf, sem):
        wid = lax.axis_index("s") * 2 + lax.axis_index("c")
        b = wid // WPB
        wq = wid % WPB
        pltpu.sync_copy(nid_hbm.at[b, wq], idx_v)
        for g in range(0, NCHUNK, FIRE):
            handles = [
                pltpu.async_copy(v_hbm.at[idx_v.at[c]],
                                 buf.at[c - g], sem)
                for c in range(g, g + FIRE)
            ]
            for h in handles:
                h.wait()
            for c in range(g, g + FIRE):
                pltpu.sync_copy(
                    buf.at[c - g],
                    out_hbm.at[b, pl.ds(wq * IDS + c * 128, 128)])

    return dens(vflat, nid_g)


def _sample_indices(vertices, neighbor_index_density):
    nid_g = (neighbor_index_density.astype(jnp.int32)
             + (jnp.arange(_BS, dtype=jnp.int32) * _V)[:, None, None])
    vpad = jnp.pad(vertices.reshape(_BS * _V, 3), ((0, 0), (0, 125)))
    neighbor_to_vertices = _dens_gather(
        vpad,
        nid_g.reshape(_BS, 4, _V * _DENS_NB // 4 // 128, 128),
    )[:, :, :3].reshape(_BS, _V, _DENS_NB, 3)
    expanded_vertices = jnp.broadcast_to(
        vertices[:, :, None, :], (_BS, _V, _DENS_NB, 3))
    squared_diff = (expanded_vertices - neighbor_to_vertices) ** 2
    distances = jnp.sqrt(jnp.sum(squared_diff, axis=3))
    summed = jnp.sum(distances, axis=2)
    mn = jnp.min(summed, axis=1, keepdims=True)
    mx = jnp.max(summed, axis=1, keepdims=True)
    nd = (summed - mn) / (mx - mn)

    edges = jnp.linspace(0.0, 1.0, _NUM_BINS + 1)
    bidx = ((nd > edges[0]).astype(jnp.int32)
            + (nd > edges[1]).astype(jnp.int32)
            + (nd > edges[2]).astype(jnp.int32))
    counts = jnp.stack(
        [jnp.sum(bidx == b, axis=1) for b in range(1, _NUM_BINS + 1)],
        axis=1).astype(jnp.float32)
    bin_samples = counts / jnp.sum(counts, axis=1, keepdims=True) * _TS
    s = jnp.floor(bin_samples).astype(jnp.int32)

    g = _gumbel_table()
    mask = bidx[:, None, :] == jnp.arange(1, _NUM_BINS + 1)[None, :, None]
    scores = jnp.where(mask, g, -jnp.inf)
    _, chosen = lax.top_k(scores, _TS)

    cnt = jnp.concatenate(
        [jnp.zeros((_BS, 1), jnp.int32), jnp.cumsum(s, axis=1)[:, :2]], axis=1)
    pos = jnp.arange(_TS)
    wpos = jnp.where(pos[None, None, :] < s[:, :, None],
                     cnt[:, :, None] + pos[None, None, :], _TS)
    bi = jnp.broadcast_to(jnp.arange(_BS)[:, None, None], wpos.shape)
    sel = (jnp.zeros((_BS, _TS), jnp.int32)
           .at[bi, wpos].set(chosen.astype(jnp.int32), mode='drop'))
    return sel


def _knn_body(dist_ref, out_ref):
    dist = dist_ref[0]
    col = lax.broadcasted_iota(jnp.int32, (_TS, _V), 1)
    inf = jnp.float32(jnp.inf)
    b = pl.program_id(0)
    for k in range(_NB + 1):
        m = jnp.min(dist, axis=1, keepdims=True)
        idx = jnp.min(jnp.where(dist == m, col, _V), axis=1, keepdims=True)
        if k > 0:
            out_ref[0, :, pl.ds(k - 1, 1)] = idx + b * _V
        dist = jnp.where(col == idx, inf, dist)


def _knn(dist):
    return pl.pallas_call(
        _knn_body,
        grid=(_BS,),
        in_specs=[
            pl.BlockSpec((1, _TS, _V), lambda b: (b, 0, 0)),
        ],
        out_specs=pl.BlockSpec((1, _TS, _NB), lambda b: (b, 0, 0)),
        out_shape=jax.ShapeDtypeStruct((_BS, _TS, _NB), jnp.int32),
    )(dist)


def _pool_gather(fm_flat, ids_flat):
    NC, NS = 2, 16
    NW = NC * NS
    ROWS = _BS * _TS
    RPW = ROWS // NW
    CH = 8
    NCH = RPW // CH
    G = CH * _NB

    mesh = plsc.VectorSubcoreMesh(core_axis_name="c", subcore_axis_name="s")

    @functools.partial(
        pl.kernel,
        mesh=mesh,
        out_type=jax.ShapeDtypeStruct((ROWS, _C), jnp.float32),
        scratch_types=[
            pltpu.VMEM((G,), jnp.int32),
            pltpu.VMEM((G, _C), jnp.float32),
            pltpu.VMEM((CH, _C), jnp.float32),
            pltpu.SemaphoreType.DMA,
        ],
    )
    def pool(fm_hbm, ids_hbm, out_hbm, idx_v, rows_v, out_v, sem):
        wid = lax.axis_index("s") * NC + lax.axis_index("c")

        def chunk_body(k, carry):
            base = (wid * RPW + k * CH) * _NB
            pltpu.sync_copy(ids_hbm.at[pl.ds(base, G)], idx_v)
            pltpu.async_copy(fm_hbm.at[idx_v], rows_v, sem).wait()

            def row_body(r, carry2):
                for cc in range(_C // 16):
                    sl = pl.ds(cc * 16, 16)
                    acc = rows_v[r * _NB, sl]
                    for j in range(1, _NB):
                        acc = jnp.maximum(acc, rows_v[r * _NB + j, sl])
                    out_v[r, sl] = acc
                return carry2

            lax.fori_loop(0, CH, row_body, 0)
            pltpu.sync_copy(out_v, out_hbm.at[pl.ds(wid * RPW + k * CH, CH)])
            return carry

        lax.fori_loop(0, NCH, chunk_body, 0)

    return pool(fm_flat, ids_flat)


def kernel(vertices, feature_map, neighbor_index_density):
    sel = _sample_indices(vertices, neighbor_index_density)
    bi = jnp.arange(_BS)[:, None]
    vertices_pool = vertices[bi, sel]
    inner = jnp.einsum('bvd,bwd->bvw', vertices, vertices)
    quadratic = jnp.sum(vertices ** 2, axis=2)
    inner_sel = inner[bi, sel]
    q_sel = quadratic[bi, sel]
    dist = inner_sel * -2 + quadratic[:, None, :] + q_sel[:, :, None]
    gids = _knn(dist)
    fm_flat = feature_map.reshape(_BS * _V, _C)
    pooled = _pool_gather(fm_flat, gids.reshape(-1))
    return vertices_pool, pooled.reshape(_BS, _TS, _C)


# speedup vs baseline: 22.3202x; 1.1294x over previous
import functools

import numpy as np
import jax
import jax.numpy as jnp
from jax import lax
from jax.experimental import pallas as pl
from jax.experimental.pallas import tpu as pltpu
from jax.experimental.pallas import tpu_sc as plsc

_BS, _V, _C, _NB, _POOL_RATE, _NUM_BINS, _DENS_NB = 8, 2048, 256, 16, 4, 3, 20
_TS = _V // _POOL_RATE


def _gumbel_table():
    skey = jax.random.key(42)
    ks = jax.vmap(lambda t: jax.random.fold_in(skey, t))(
        jnp.arange(_BS * _NUM_BINS))
    gs = jax.vmap(lambda k: jax.random.gumbel(k, (_V,)))(ks)
    return gs.reshape(_BS, _NUM_BINS, _V)


def _dens_gather(vflat, nid_g):
    WPB = 4
    VPW = _V // WPB
    IDS = VPW * _DENS_NB
    NCHUNK = IDS // 128
    FIRE = 4

    mesh = plsc.VectorSubcoreMesh(core_axis_name="c", subcore_axis_name="s")

    @functools.partial(
        pl.kernel, mesh=mesh,
        out_type=jax.ShapeDtypeStruct((_BS, _V * _DENS_NB, 128), jnp.float32),
        scratch_types=[
            pltpu.VMEM((NCHUNK, 128), jnp.int32),
            pltpu.VMEM((FIRE, 128, 128), jnp.float32),
            pltpu.SemaphoreType.DMA,
        ],
    )
    def dens(v_hbm, nid_hbm, out_hbm, idx_v, buf, sem):
        wid = lax.axis_index("s") * 2 + lax.axis_index("c")
        b = wid // WPB
        wq = wid % WPB
        pltpu.sync_copy(nid_hbm.at[b, wq], idx_v)
        for g in range(0, NCHUNK, FIRE):
            handles = [
                pltpu.async_copy(v_hbm.at[idx_v.at[c]],
                                 buf.at[c - g], sem)
                for c in range(g, g + FIRE)
            ]
            for h in handles:
                h.wait()
            for c in range(g, g + FIRE):
                pltpu.sync_copy(
                    buf.at[c - g],
                    out_hbm.at[b, pl.ds(wq * IDS + c * 128, 128)])

    return dens(vflat, nid_g)


def _sample_indices(vertices, neighbor_index_density):
    nid_g = (neighbor_index_density.astype(jnp.int32)
             + (jnp.arange(_BS, dtype=jnp.int32) * _V)[:, None, None])
    vpad = jnp.pad(vertices.reshape(_BS * _V, 3), ((0, 0), (0, 125)))
    neighbor_to_vertices = _dens_gather(
        vpad,
        nid_g.reshape(_BS, 4, _V * _DENS_NB // 4 // 128, 128),
    )[:, :, :3].reshape(_BS, _V, _DENS_NB, 3)
    expanded_vertices = jnp.broadcast_to(
        vertices[:, :, None, :], (_BS, _V, _DENS_NB, 3))
    squared_diff = (expanded_vertices - neighbor_to_vertices) ** 2
    distances = jnp.sqrt(jnp.sum(squared_diff, axis=3))
    summed = jnp.sum(distances, axis=2)
    mn = jnp.min(summed, axis=1, keepdims=True)
    mx = jnp.max(summed, axis=1, keepdims=True)
    nd = (summed - mn) / (mx - mn)

    edges = jnp.linspace(0.0, 1.0, _NUM_BINS + 1)
    bidx = ((nd > edges[0]).astype(jnp.int32)
            + (nd > edges[1]).astype(jnp.int32)
            + (nd > edges[2]).astype(jnp.int32))
    counts = jnp.stack(
        [jnp.sum(bidx == b, axis=1) for b in range(1, _NUM_BINS + 1)],
        axis=1).astype(jnp.float32)
    bin_samples = counts / jnp.sum(counts, axis=1, keepdims=True) * _TS
    s = jnp.floor(bin_samples).astype(jnp.int32)

    g = _gumbel_table()
    mask = bidx[:, None, :] == jnp.arange(1, _NUM_BINS + 1)[None, :, None]
    scores = jnp.where(mask, g, -jnp.inf)
    _, chosen = lax.top_k(scores, _TS)

    cnt = jnp.concatenate(
        [jnp.zeros((_BS, 1), jnp.int32), jnp.cumsum(s, axis=1)[:, :2]], axis=1)
    pos = jnp.arange(_TS)
    wpos = jnp.where(pos[None, None, :] < s[:, :, None],
                     cnt[:, :, None] + pos[None, None, :], _TS)
    bi = jnp.broadcast_to(jnp.arange(_BS)[:, None, None], wpos.shape)
    sel = (jnp.zeros((_BS, _TS), jnp.int32)
           .at[bi, wpos].set(chosen.astype(jnp.int32), mode='drop'))
    return sel


def _knn_body(dist_ref, out_ref):
    dist = dist_ref[0]
    col = lax.broadcasted_iota(jnp.int32, (_TS, _V), 1)
    inf = jnp.float32(jnp.inf)
    b = pl.program_id(0)
    for k in range(_NB + 1):
        m = jnp.min(dist, axis=1, keepdims=True)
        idx = jnp.min(jnp.where(dist == m, col, _V), axis=1, keepdims=True)
        if k > 0:
            out_ref[0, :, pl.ds(k - 1, 1)] = idx + b * _V
        dist = jnp.where(col == idx, inf, dist)


def _knn(dist):
    return pl.pallas_call(
        _knn_body,
        grid=(_BS,),
        in_specs=[
            pl.BlockSpec((1, _TS, _V), lambda b: (b, 0, 0)),
        ],
        out_specs=pl.BlockSpec((1, _TS, _NB), lambda b: (b, 0, 0)),
        out_shape=jax.ShapeDtypeStruct((_BS, _TS, _NB), jnp.int32),
    )(dist)


def _pool_gather(fm_flat, ids_flat):
    NC, NS = 2, 16
    NW = NC * NS
    ROWS = _BS * _TS
    RPW = ROWS // NW
    CH = 8
    NCH = RPW // CH
    G = CH * _NB

    mesh = plsc.VectorSubcoreMesh(core_axis_name="c", subcore_axis_name="s")

    @functools.partial(
        pl.kernel,
        mesh=mesh,
        out_type=jax.ShapeDtypeStruct((ROWS, _C), jnp.float32),
        scratch_types=[
            pltpu.VMEM((G,), jnp.int32),
            pltpu.VMEM((G, _C), jnp.float32),
            pltpu.VMEM((CH, _C), jnp.float32),
            pltpu.SemaphoreType.DMA,
        ],
    )
    def pool(fm_hbm, ids_hbm, out_hbm, idx_v, rows_v, out_v, sem):
        wid = lax.axis_index("s") * NC + lax.axis_index("c")

        def chunk_body(k, carry):
            base = (wid * RPW + k * CH) * _NB
            pltpu.sync_copy(ids_hbm.at[pl.ds(base, G)], idx_v)
            pltpu.async_copy(fm_hbm.at[idx_v], rows_v, sem).wait()

            def row_body(r, carry2):
                for cc in range(_C // 16):
                    sl = pl.ds(cc * 16, 16)
                    acc = rows_v[r * _NB, sl]
                    for j in range(1, _NB):
                        acc = jnp.maximum(acc, rows_v[r * _NB + j, sl])
                    out_v[r, sl] = acc
                return carry2

            lax.fori_loop(0, CH, row_body, 0)
            pltpu.sync_copy(out_v, out_hbm.at[pl.ds(wid * RPW + k * CH, CH)])
            return carry

        lax.fori_loop(0, NCH, chunk_body, 0)

    return pool(fm_flat, ids_flat)


def kernel(vertices, feature_map, neighbor_index_density):
    sel = _sample_indices(vertices, neighbor_index_density)
    bi = jnp.arange(_BS)[:, None]
    vertices_pool = vertices[bi, sel]
    inner = jnp.einsum('bvd,bwd->bvw', vertices, vertices)
    quadratic = jnp.sum(vertices ** 2, axis=2)
    inner_sel = inner[bi, sel]
    q_sel = quadratic[bi, sel]
    dist = inner_sel * -2 + quadratic[:, None, :] + q_sel[:, :, None]
    gids = _knn(dist)
    fm_flat = feature_map.reshape(_BS * _V, _C)
    pooled = _pool_gather(fm_flat, gids.reshape(-1))
    return vertices_pool, pooled.reshape(_BS, _TS, _C)
